# initial kernel scaffold (unmeasured)
import jax
import jax.numpy as jnp
from jax import lax
from jax.experimental import pallas as pl
from jax.experimental.pallas import tpu as pltpu

N_DEV = 4
M = 4096
N_OUT = 2048
CH = M // N_DEV
HALF = N_OUT // 2


def _ar_body(p_ref, o_ref, recv_buf, acc_buf, stage_buf,
             rs_send, rs_recv, ag_send, ag_recv, cp_sems, st_sems):
    i = lax.axis_index("i")
    right = lax.rem(i + 1, N_DEV)
    left = lax.rem(i + N_DEV - 1, N_DEV)

    bar = pltpu.get_barrier_semaphore()
    for nbr in (left, right):
        pl.semaphore_signal(bar, inc=1, device_id=(nbr,),
                            device_id_type=pl.DeviceIdType.MESH)
    pl.semaphore_wait(bar, 2)

    def cmod(v):
        return lax.rem(v + 2 * N_DEV, N_DEV)

    dirs = ((0, 1, right), (1, -1, left))

    store_copies = []
    for s in range(N_DEV - 1):
        rdmas = []
        copies = []
        for d, sgn, tgt in dirs:
            c_send = cmod(i - sgn * (s + 1))
            c_recv = cmod(i - sgn * (s + 2))
            if s == 0:
                src = p_ref.at[pl.ds(c_send * CH, CH), pl.ds(d * HALF, HALF)]
            else:
                src = acc_buf.at[d, (s - 1) % 2]
            rdma = pltpu.make_async_remote_copy(
                src_ref=src,
                dst_ref=recv_buf.at[d, s],
                send_sem=rs_send.at[d, s],
                recv_sem=rs_recv.at[d, s],
                device_id=(tgt,),
                device_id_type=pl.DeviceIdType.MESH,
            )
            rdma.start()
            cp = pltpu.make_async_copy(
                p_ref.at[pl.ds(c_recv * CH, CH), pl.ds(d * HALF, HALF)],
                stage_buf.at[d, s % 2],
                cp_sems.at[d, s % 2],
            )
            cp.start()
            rdmas.append(rdma)
            copies.append(cp)
        for (d, sgn, tgt), rdma, cp in zip(dirs, rdmas, copies):
            rdma.wait_send()
            rdma.wait_recv()
            cp.wait()
            total = recv_buf[d, s] + stage_buf[d, s % 2]
            if s < N_DEV - 2:
                acc_buf[d, s % 2] = total
            else:
                acc_buf[d, 0] = jax.nn.silu(total)
                st = pltpu.make_async_copy(
                    acc_buf.at[d, 0],
                    o_ref.at[pl.ds(i * CH, CH), pl.ds(d * HALF, HALF)],
                    st_sems.at[d],
                )
                st.start()
                store_copies.append(st)
    for st in store_copies:
        st.wait()

    for s in range(N_DEV - 1):
        rdmas = []
        for d, sgn, tgt in dirs:
            c_send = cmod(i - sgn * s)
            sl_rows = pl.ds(c_send * CH, CH)
            sl_cols = pl.ds(d * HALF, HALF)
            rdma = pltpu.make_async_remote_copy(
                src_ref=o_ref.at[sl_rows, sl_cols],
                dst_ref=o_ref.at[sl_rows, sl_cols],
                send_sem=ag_send.at[d, s],
                recv_sem=ag_recv.at[d, s],
                device_id=(tgt,),
                device_id_type=pl.DeviceIdType.MESH,
            )
            rdma.start()
            rdmas.append(rdma)
        for rdma in rdmas:
            rdma.wait_send()
            rdma.wait_recv()


def _allreduce_silu(partial):
    return pl.pallas_call(
        _ar_body,
        out_shape=jax.ShapeDtypeStruct((M, N_OUT), jnp.float32),
        in_specs=[pl.BlockSpec(memory_space=pltpu.ANY)],
        out_specs=pl.BlockSpec(memory_space=pltpu.ANY),
        scratch_shapes=[
            pltpu.VMEM((2, N_DEV - 1, CH, HALF), jnp.float32),
            pltpu.VMEM((2, 2, CH, HALF), jnp.float32),
            pltpu.VMEM((2, 2, CH, HALF), jnp.float32),
            pltpu.SemaphoreType.DMA((2, N_DEV - 1)),
            pltpu.SemaphoreType.DMA((2, N_DEV - 1)),
            pltpu.SemaphoreType.DMA((2, N_DEV - 1)),
            pltpu.SemaphoreType.DMA((2, N_DEV - 1)),
            pltpu.SemaphoreType.DMA((2, 2)),
            pltpu.SemaphoreType.DMA((2,)),
        ],
        compiler_params=pltpu.CompilerParams(collective_id=0),
    )(partial)


def kernel(x, w_mat):
    partial = jnp.dot(
        x.astype(jnp.bfloat16),
        w_mat.astype(jnp.bfloat16),
        preferred_element_type=jnp.float32,
    )
    return _allreduce_silu(partial)


# baseline (device time: 340854 ns/iter reference)
import jax
import jax.numpy as jnp
from jax import lax
from jax.experimental import pallas as pl
from jax.experimental.pallas import tpu as pltpu

N_DEV = 4
M = 4096
N_OUT = 2048
CH = M // N_DEV
HALF = N_OUT // 2


def _ar_body(p_ref, o_ref, recv_buf, stage_buf,
             rs_send, rs_recv, ag_send, ag_recv, cp_sems, st_sems):
    i = lax.axis_index("i")
    right = lax.rem(i + 1, N_DEV)
    left = lax.rem(i + N_DEV - 1, N_DEV)

    bar = pltpu.get_barrier_semaphore()
    for nbr in (left, right):
        pl.semaphore_signal(bar, inc=1, device_id=(nbr,),
                            device_id_type=pl.DeviceIdType.MESH)
    pl.semaphore_wait(bar, 2)

    def cmod(v):
        return lax.rem(v + 2 * N_DEV, N_DEV)

    dirs = ((0, 1, right), (1, -1, left))

    store_copies = []
    for s in range(N_DEV - 1):
        rdmas = []
        copies = []
        for d, sgn, tgt in dirs:
            c_send = cmod(i - sgn * (s + 1))
            c_recv = cmod(i - sgn * (s + 2))
            if s == 0:
                src = p_ref.at[pl.ds(c_send * CH, CH), pl.ds(d * HALF, HALF)]
            else:
                src = recv_buf.at[d, (s - 1) % 2]
            rdma = pltpu.make_async_remote_copy(
                src_ref=src,
                dst_ref=recv_buf.at[d, s % 2],
                send_sem=rs_send.at[d, s % 2],
                recv_sem=rs_recv.at[d, s % 2],
                device_id=(tgt,),
                device_id_type=pl.DeviceIdType.MESH,
            )
            rdma.start()
            cp = pltpu.make_async_copy(
                p_ref.at[pl.ds(c_recv * CH, CH), pl.ds(d * HALF, HALF)],
                stage_buf.at[d],
                cp_sems.at[d],
            )
            cp.start()
            rdmas.append(rdma)
            copies.append(cp)
        for (d, sgn, tgt), rdma, cp in zip(dirs, rdmas, copies):
            rdma.wait_send()
            rdma.wait_recv()
            cp.wait()
            total = recv_buf[d, s % 2] + stage_buf[d]
            if s < N_DEV - 2:
                recv_buf[d, s % 2] = total
            else:
                recv_buf[d, s % 2] = jax.nn.silu(total)
                st = pltpu.make_async_copy(
                    recv_buf.at[d, s % 2],
                    o_ref.at[pl.ds(i * CH, CH), pl.ds(d * HALF, HALF)],
                    st_sems.at[d],
                )
                st.start()
                store_copies.append(st)
    for st in store_copies:
        st.wait()

    for s in range(N_DEV - 1):
        rdmas = []
        for d, sgn, tgt in dirs:
            c_send = cmod(i - sgn * s)
            sl_rows = pl.ds(c_send * CH, CH)
            sl_cols = pl.ds(d * HALF, HALF)
            rdma = pltpu.make_async_remote_copy(
                src_ref=o_ref.at[sl_rows, sl_cols],
                dst_ref=o_ref.at[sl_rows, sl_cols],
                send_sem=ag_send.at[d, s],
                recv_sem=ag_recv.at[d, s],
                device_id=(tgt,),
                device_id_type=pl.DeviceIdType.MESH,
            )
            rdma.start()
            rdmas.append(rdma)
        for rdma in rdmas:
            rdma.wait_send()
            rdma.wait_recv()


def _allreduce_silu(partial):
    return pl.pallas_call(
        _ar_body,
        out_shape=jax.ShapeDtypeStruct((M, N_OUT), jnp.float32),
        in_specs=[pl.BlockSpec(memory_space=pl.ANY)],
        out_specs=pl.BlockSpec(memory_space=pl.ANY),
        scratch_shapes=[
            pltpu.VMEM((2, 2, CH, HALF), jnp.float32),
            pltpu.VMEM((2, CH, HALF), jnp.float32),
            pltpu.SemaphoreType.DMA((2, 2)),
            pltpu.SemaphoreType.DMA((2, 2)),
            pltpu.SemaphoreType.DMA((2, N_DEV - 1)),
            pltpu.SemaphoreType.DMA((2, N_DEV - 1)),
            pltpu.SemaphoreType.DMA((2,)),
            pltpu.SemaphoreType.DMA((2,)),
        ],
        compiler_params=pltpu.CompilerParams(collective_id=0),
    )(partial)


def kernel(x, w_mat):
    partial = jnp.dot(
        x.astype(jnp.bfloat16),
        w_mat.astype(jnp.bfloat16),
        preferred_element_type=jnp.float32,
    )
    return _allreduce_silu(partial)


# device time: 328728 ns/iter; 1.0369x vs baseline; 1.0369x over previous
import jax
import jax.numpy as jnp
from jax import lax
from jax.experimental import pallas as pl
from jax.experimental.pallas import tpu as pltpu

N_DEV = 4
M = 4096
K = 1024
N_OUT = 2048
CH = M // N_DEV
HALF = N_OUT // 2


def _body(x_ref, wb_ref, o_ref, w_v, x_v, part, recv,
          sem_w, sem_x, rs_send, rs_recv, ag_send, ag_recv, st_sems):
    i = lax.axis_index("i")
    right = lax.rem(i + 1, N_DEV)
    left = lax.rem(i + N_DEV - 1, N_DEV)

    bar = pltpu.get_barrier_semaphore()
    for nbr in (left, right):
        pl.semaphore_signal(bar, inc=1, device_id=(nbr,),
                            device_id_type=pl.DeviceIdType.MESH)
    pl.semaphore_wait(bar, 2)

    def cmod(v):
        return lax.rem(v + 2 * N_DEV, N_DEV)

    dirs = ((0, 1, right), (1, -1, left))

    chunks = [cmod(i - 1), cmod(i + 1), cmod(i + 2), i]
    x_cps = [
        pltpu.make_async_copy(
            x_ref.at[pl.ds(chunks[k] * CH, CH), :], x_v.at[k % 2],
            sem_x.at[k % 2])
        for k in range(4)
    ]

    w_cp = pltpu.make_async_copy(wb_ref, w_v, sem_w)
    w_cp.start()
    x_cps[0].start()
    w_cp.wait()
    x_cps[0].wait()
    x_cps[1].start()

    def gemm(k, slot):
        part[slot] = jnp.dot(
            x_v[k % 2].astype(jnp.bfloat16), w_v[...],
            preferred_element_type=jnp.float32)

    gemm(0, 0)
    x_cps[1].wait()
    x_cps[2].start()
    gemm(1, 1)

    rs0 = []
    for d, sgn, tgt in dirs:
        rdma = pltpu.make_async_remote_copy(
            src_ref=part.at[d, :, pl.ds(d * HALF, HALF)],
            dst_ref=recv.at[d, 0],
            send_sem=rs_send.at[d, 0], recv_sem=rs_recv.at[d, 0],
            device_id=(tgt,), device_id_type=pl.DeviceIdType.MESH)
        rdma.start()
        rs0.append(rdma)

    x_cps[2].wait()
    x_cps[3].start()
    gemm(2, 2)

    for (d, sgn, tgt), rdma in zip(dirs, rs0):
        rdma.wait_recv()
        recv[d, 0] = recv[d, 0] + part[2, :, d * HALF:(d + 1) * HALF]
    for rdma in rs0:
        rdma.wait_send()

    rs1 = []
    for d, sgn, tgt in dirs:
        rdma = pltpu.make_async_remote_copy(
            src_ref=recv.at[d, 0], dst_ref=recv.at[d, 1],
            send_sem=rs_send.at[d, 1], recv_sem=rs_recv.at[d, 1],
            device_id=(tgt,), device_id_type=pl.DeviceIdType.MESH)
        rdma.start()
        rs1.append(rdma)

    x_cps[3].wait()
    gemm(3, 2)

    adds1 = (part.at[1, :, 0:HALF], part.at[0, :, HALF:N_OUT])
    for (d, sgn, tgt), rdma in zip(dirs, rs1):
        rdma.wait_recv()
        recv[d, 1] = recv[d, 1] + adds1[d][...]
    for rdma in rs1:
        rdma.wait_send()

    rs2 = []
    for d, sgn, tgt in dirs:
        rdma = pltpu.make_async_remote_copy(
            src_ref=recv.at[d, 1], dst_ref=recv.at[d, 0],
            send_sem=rs_send.at[d, 0], recv_sem=rs_recv.at[d, 0],
            device_id=(tgt,), device_id_type=pl.DeviceIdType.MESH)
        rdma.start()
        rs2.append(rdma)
    for (d, sgn, tgt), rdma in zip(dirs, rs2):
        rdma.wait_recv()
        recv[d, 0] = jax.nn.silu(
            recv[d, 0] + part[2, :, d * HALF:(d + 1) * HALF])
    for rdma in rs2:
        rdma.wait_send()

    sts = []
    for d, _, _ in dirs:
        st = pltpu.make_async_copy(
            recv.at[d, 0],
            o_ref.at[pl.ds(i * CH, CH), pl.ds(d * HALF, HALF)],
            st_sems.at[d])
        st.start()
        sts.append(st)

    for s in range(N_DEV - 1):
        rdmas = []
        for d, sgn, tgt in dirs:
            c_send = cmod(i - sgn * s)
            dst = o_ref.at[pl.ds(c_send * CH, CH), pl.ds(d * HALF, HALF)]
            src = recv.at[d, 0] if s == 0 else dst
            rdma = pltpu.make_async_remote_copy(
                src_ref=src, dst_ref=dst,
                send_sem=ag_send.at[d, s], recv_sem=ag_recv.at[d, s],
                device_id=(tgt,), device_id_type=pl.DeviceIdType.MESH)
            rdma.start()
            rdmas.append(rdma)
        for rdma in rdmas:
            rdma.wait_recv()
        for rdma in rdmas:
            rdma.wait_send()
    for st in sts:
        st.wait()


def kernel(x, w_mat):
    wb = w_mat.astype(jnp.bfloat16)
    return pl.pallas_call(
        _body,
        out_shape=jax.ShapeDtypeStruct((M, N_OUT), jnp.float32),
        in_specs=[pl.BlockSpec(memory_space=pl.ANY),
                  pl.BlockSpec(memory_space=pl.ANY)],
        out_specs=pl.BlockSpec(memory_space=pl.ANY),
        scratch_shapes=[
            pltpu.VMEM((K, N_OUT), jnp.bfloat16),
            pltpu.VMEM((2, CH, K), jnp.float32),
            pltpu.VMEM((3, CH, N_OUT), jnp.float32),
            pltpu.VMEM((2, 2, CH, HALF), jnp.float32),
            pltpu.SemaphoreType.DMA,
            pltpu.SemaphoreType.DMA((2,)),
            pltpu.SemaphoreType.DMA((2, 2)),
            pltpu.SemaphoreType.DMA((2, 2)),
            pltpu.SemaphoreType.DMA((2, N_DEV - 1)),
            pltpu.SemaphoreType.DMA((2, N_DEV - 1)),
            pltpu.SemaphoreType.DMA((2,)),
        ],
        compiler_params=pltpu.CompilerParams(
            collective_id=0,
            vmem_limit_bytes=64 * 1024 * 1024,
        ),
    )(x, wb)


# device time: 326311 ns/iter; 1.0446x vs baseline; 1.0074x over previous
import jax
import jax.numpy as jnp
from jax import lax
from jax.experimental import pallas as pl
from jax.experimental.pallas import tpu as pltpu

N_DEV = 4
M = 4096
K = 1024
N_OUT = 2048
CH = M // N_DEV
HALF = N_OUT // 2


def _body(x_ref, w_ref, o_ref, w_v, x_v, part, recv,
          sem_x, rs_send, rs_recv, ag_send, ag_recv, st_sems):
    i = lax.axis_index("i")
    right = lax.rem(i + 1, N_DEV)
    left = lax.rem(i + N_DEV - 1, N_DEV)

    bar = pltpu.get_barrier_semaphore()
    for nbr in (left, right):
        pl.semaphore_signal(bar, inc=1, device_id=(nbr,),
                            device_id_type=pl.DeviceIdType.MESH)
    pl.semaphore_wait(bar, 2)

    def cmod(v):
        return lax.rem(v + 2 * N_DEV, N_DEV)

    dirs = ((0, 1, right), (1, -1, left))

    chunks = [cmod(i - 1), cmod(i + 1), cmod(i + 2), i]
    x_cps = [
        pltpu.make_async_copy(
            x_ref.at[pl.ds(chunks[k] * CH, CH), :], x_v.at[k % 2],
            sem_x.at[k % 2])
        for k in range(4)
    ]

    w_cps = [
        pltpu.make_async_copy(
            w_ref.at[:, pl.ds(h * HALF, HALF)], x_v.at[h], sem_x.at[h])
        for h in range(2)
    ]
    for cp in w_cps:
        cp.start()
    for h, cp in enumerate(w_cps):
        cp.wait()
        w_v[:, h * HALF:(h + 1) * HALF] = x_v[h].astype(jnp.bfloat16)

    x_cps[0].start()
    x_cps[0].wait()
    x_cps[1].start()

    def gemm(k, slot):
        part[slot] = jnp.dot(
            x_v[k % 2].astype(jnp.bfloat16), w_v[...],
            preferred_element_type=jnp.float32)

    gemm(0, 0)
    x_cps[1].wait()
    x_cps[2].start()
    gemm(1, 1)

    rs0 = []
    for d, sgn, tgt in dirs:
        rdma = pltpu.make_async_remote_copy(
            src_ref=part.at[d, :, pl.ds(d * HALF, HALF)],
            dst_ref=recv.at[d, 0],
            send_sem=rs_send.at[d, 0], recv_sem=rs_recv.at[d, 0],
            device_id=(tgt,), device_id_type=pl.DeviceIdType.MESH)
        rdma.start()
        rs0.append(rdma)

    x_cps[2].wait()
    x_cps[3].start()
    gemm(2, 2)

    for (d, sgn, tgt), rdma in zip(dirs, rs0):
        rdma.wait_recv()
        recv[d, 0] = recv[d, 0] + part[2, :, d * HALF:(d + 1) * HALF]
    for rdma in rs0:
        rdma.wait_send()

    rs1 = []
    for d, sgn, tgt in dirs:
        rdma = pltpu.make_async_remote_copy(
            src_ref=recv.at[d, 0], dst_ref=recv.at[d, 1],
            send_sem=rs_send.at[d, 1], recv_sem=rs_recv.at[d, 1],
            device_id=(tgt,), device_id_type=pl.DeviceIdType.MESH)
        rdma.start()
        rs1.append(rdma)

    x_cps[3].wait()
    gemm(3, 2)

    adds1 = (part.at[1, :, 0:HALF], part.at[0, :, HALF:N_OUT])
    for (d, sgn, tgt), rdma in zip(dirs, rs1):
        rdma.wait_recv()
        recv[d, 1] = recv[d, 1] + adds1[d][...]
    for rdma in rs1:
        rdma.wait_send()

    rs2 = []
    for d, sgn, tgt in dirs:
        rdma = pltpu.make_async_remote_copy(
            src_ref=recv.at[d, 1], dst_ref=recv.at[d, 0],
            send_sem=rs_send.at[d, 0], recv_sem=rs_recv.at[d, 0],
            device_id=(tgt,), device_id_type=pl.DeviceIdType.MESH)
        rdma.start()
        rs2.append(rdma)
    for (d, sgn, tgt), rdma in zip(dirs, rs2):
        rdma.wait_recv()
        recv[d, 0] = jax.nn.silu(
            recv[d, 0] + part[2, :, d * HALF:(d + 1) * HALF])
    for rdma in rs2:
        rdma.wait_send()

    sts = []
    for d, _, _ in dirs:
        st = pltpu.make_async_copy(
            recv.at[d, 0],
            o_ref.at[pl.ds(i * CH, CH), pl.ds(d * HALF, HALF)],
            st_sems.at[d])
        st.start()
        sts.append(st)

    for s in range(N_DEV - 1):
        rdmas = []
        for d, sgn, tgt in dirs:
            c_send = cmod(i - sgn * s)
            dst = o_ref.at[pl.ds(c_send * CH, CH), pl.ds(d * HALF, HALF)]
            src = recv.at[d, 0] if s == 0 else dst
            rdma = pltpu.make_async_remote_copy(
                src_ref=src, dst_ref=dst,
                send_sem=ag_send.at[d, s], recv_sem=ag_recv.at[d, s],
                device_id=(tgt,), device_id_type=pl.DeviceIdType.MESH)
            rdma.start()
            rdmas.append(rdma)
        for rdma in rdmas:
            rdma.wait_recv()
        for rdma in rdmas:
            rdma.wait_send()
    for st in sts:
        st.wait()


def kernel(x, w_mat):
    return pl.pallas_call(
        _body,
        out_shape=jax.ShapeDtypeStruct((M, N_OUT), jnp.float32),
        in_specs=[pl.BlockSpec(memory_space=pl.ANY),
                  pl.BlockSpec(memory_space=pl.ANY)],
        out_specs=pl.BlockSpec(memory_space=pl.ANY),
        scratch_shapes=[
            pltpu.VMEM((K, N_OUT), jnp.bfloat16),
            pltpu.VMEM((2, CH, K), jnp.float32),
            pltpu.VMEM((3, CH, N_OUT), jnp.float32),
            pltpu.VMEM((2, 2, CH, HALF), jnp.float32),
            pltpu.SemaphoreType.DMA((2,)),
            pltpu.SemaphoreType.DMA((2, 2)),
            pltpu.SemaphoreType.DMA((2, 2)),
            pltpu.SemaphoreType.DMA((2, N_DEV - 1)),
            pltpu.SemaphoreType.DMA((2, N_DEV - 1)),
            pltpu.SemaphoreType.DMA((2,)),
        ],
        compiler_params=pltpu.CompilerParams(
            collective_id=0,
            vmem_limit_bytes=64 * 1024 * 1024,
        ),
    )(x, w_mat)
